# baseline (speedup 1.0000x reference)
import functools

import jax
import jax.numpy as jnp
from jax import lax
from jax.experimental import pallas as pl
from jax.experimental.pallas import tpu as pltpu
from jax.experimental.pallas import tpu_sc as plsc

NUM_CORES = 2
NUM_SUBCORES = 16
NUM_WORKERS = NUM_CORES * NUM_SUBCORES
L = 16


def _diagonals():
    iota = jnp.arange(L, dtype=jnp.int32)
    e = [lax.rem(iota + d, L) for d in range(L)]
    return iota, e


def _repack_tile(src, dst, iota, e, pe, je, rbl, cb):
    c_vec = cb * L + iota
    r_vecs = [rbl * L + e[d] for d in range(L)]
    vs = [plsc.load_gather(src, [c_vec, r_vecs[d]]) for d in range(L)]
    for d in range(L):
        plsc.store_scatter(
            dst, [rbl * (L // 2) + pe[d], c_vec + je[d]], vs[d]
        )


def _repack_kernel(V, D):
    RB = 4 * D
    n_blocks = V // RB
    steps = -(-n_blocks // NUM_WORKERS)
    steps += steps % 2
    mesh = plsc.VectorSubcoreMesh(core_axis_name="c", subcore_axis_name="s")

    @functools.partial(
        pl.kernel,
        mesh=mesh,
        out_type=jax.ShapeDtypeStruct((V // 2, 2 * D), jnp.float32),
        scratch_types=[
            pltpu.VMEM((2, D, RB), jnp.float32),
            pltpu.VMEM((2, RB // 2, 2 * D), jnp.float32),
        ]
        + [pltpu.SemaphoreType.DMA] * 4,
        compiler_params=pltpu.CompilerParams(needs_layout_passes=False),
    )
    def body(tt_hbm, tail_hbm, out_hbm, tin, tout, *sems):
        gsem = sems[:2]
        ssem = sems[2:]
        wid = lax.axis_index("s") * NUM_CORES + lax.axis_index("c")

        def block_id(k):
            return jnp.minimum(wid + k * NUM_WORKERS, n_blocks - 1)

        def load(k, b):
            r0 = pl.multiple_of(block_id(k) * RB, RB)
            return pltpu.make_async_copy(
                tt_hbm.at[:, pl.ds(r0, RB)], tin.at[b], gsem[b]
            )

        def store(k, b):
            p0 = pl.multiple_of(block_id(k) * (RB // 2), RB // 2)
            return pltpu.make_async_copy(
                tout.at[b], out_hbm.at[pl.ds(p0, RB // 2)], ssem[b]
            )

        load(0, 0).start()
        load(1, 1).start()

        iota, e = _diagonals()
        pe = [lax.shift_right_logical(e[d], 1) for d in range(L)]
        je = [lax.shift_left(jnp.bitwise_and(e[d], 1), 6) for d in range(L)]

        def step(i, _):
            for b in range(2):
                k = 2 * i + b
                load(k, b).wait()

                @pl.when(i > 0)
                def _():
                    store(k - 2, b).wait()

                def tile(t, _):
                    cb = t & (D // L - 1)
                    rbl = lax.shift_right_logical(t, 2)
                    _repack_tile(
                        tin.at[b], tout.at[b], iota, e, pe, je, rbl, cb
                    )
                    return 0

                lax.fori_loop(0, (D // L) * (RB // L), tile, 0)

                @pl.when(k + 2 < steps)
                def _():
                    load(k + 2, b).start()

                store(k, b).start()
            return 0

        lax.fori_loop(0, steps // 2, step, 0)
        store(steps - 2, 0).wait()
        store(steps - 1, 1).wait()

        @pl.when(wid == 0)
        def _():
            stage = tin.at[0, :, pl.ds(0, 2 * D)]
            pltpu.sync_copy(tail_hbm, stage)
            pltpu.sync_copy(stage, out_hbm.at[pl.ds(V // 2 - D, D)])

    return body


def _lookup_kernel(B0, B1, V, D):
    W = B0 // NUM_WORKERS
    NB = 4
    mesh = plsc.VectorSubcoreMesh(core_axis_name="c", subcore_axis_name="s")

    @functools.partial(
        pl.kernel,
        mesh=mesh,
        out_type=jax.ShapeDtypeStruct((B1, D, B0), jnp.float32),
        scratch_types=[
            pltpu.VMEM((B1, W), jnp.int32),
            pltpu.VMEM((NB, W), jnp.int32),
            pltpu.VMEM((NB, W), jnp.int32),
            pltpu.VMEM((NB, W, 2 * D), jnp.float32),
            pltpu.VMEM((2, D, W), jnp.float32),
        ]
        + [pltpu.SemaphoreType.DMA] * (NB + 2),
        compiler_params=pltpu.CompilerParams(needs_layout_passes=False),
    )
    def body(idx_hbm, tbl_hbm, out_hbm, idx_v, pidx, h64_v, grows, tout,
             *sems):
        gsem = sems[:NB]
        ssem = sems[NB:]
        wid = lax.axis_index("s") * NUM_CORES + lax.axis_index("c")
        c0 = pl.multiple_of(wid * W, W)
        pltpu.sync_copy(idx_hbm.at[:, pl.ds(c0, W)], idx_v)

        def prep(s, b):
            for g in range(W // L):
                r = idx_v[s, pl.ds(L * g, L)]
                pidx.at[b][pl.ds(L * g, L)] = lax.shift_right_logical(r, 1)
                h64_v.at[b][pl.ds(L * g, L)] = lax.shift_left(
                    jnp.bitwise_and(r, 1), 6
                )

        def gather(b):
            return pltpu.make_async_copy(
                tbl_hbm.at[pidx.at[b]], grows.at[b], gsem[b]
            )

        def start_gather(s, b):
            prep(s, b)
            gather(b).start()

        def store(s, b):
            return pltpu.make_async_copy(
                tout.at[b], out_hbm.at[s, :, pl.ds(c0, W)], ssem[b]
            )

        for b in range(NB):
            start_gather(b, b)

        iota, e = _diagonals()

        def step(i, _):
            for b in range(NB):
                s = NB * i + b
                tb = b % 2
                gather(b).wait()

                if b >= 2:
                    store(s - 2, tb).wait()
                else:

                    @pl.when(i > 0)
                    def _():
                        store(s - 2, tb).wait()

                def tile(t, _):
                    q = t & (D // L - 1)
                    g = lax.shift_right_logical(t, 2)
                    c_vec = q * L + iota
                    r_vecs = [g * L + e[d] for d in range(L)]
                    hs = [
                        plsc.load_gather(h64_v.at[b], [r_vecs[d]])
                        for d in range(L)
                    ]
                    vs = [
                        plsc.load_gather(
                            grows.at[b], [r_vecs[d], c_vec + hs[d]]
                        )
                        for d in range(L)
                    ]
                    for d in range(L):
                        plsc.store_scatter(
                            tout.at[tb], [c_vec, r_vecs[d]], vs[d]
                        )
                    return 0

                lax.fori_loop(0, (D // L) * (W // L), tile, 0)

                @pl.when(s + NB < B1)
                def _():
                    start_gather(s + NB, b)

                store(s, tb).start()
            return 0

        lax.fori_loop(0, B1 // NB, step, 0)
        store(B1 - 2, 0).wait()
        store(B1 - 1, 1).wait()

    return body


def kernel(indices, table):
    B0, B1 = indices.shape
    V, D = table.shape
    tt = table.T
    tail = table[V - 2 * D :].reshape(D, 2 * D)
    tbl = _repack_kernel(V, D)(tt, tail)
    idx2 = indices.T.astype(jnp.int32)
    out2 = _lookup_kernel(B0, B1, V, D)(idx2, tbl)
    return out2.transpose(2, 0, 1)


# speedup vs baseline: 1.5819x; 1.5819x over previous
import functools

import jax
import jax.numpy as jnp
from jax import lax
from jax.experimental import pallas as pl
from jax.experimental.pallas import tpu as pltpu
from jax.experimental.pallas import tpu_sc as plsc

NUM_CORES = 2
NUM_SUBCORES = 16
NUM_WORKERS = NUM_CORES * NUM_SUBCORES
L = 16


def _diagonals():
    iota = jnp.arange(L, dtype=jnp.int32)
    e = [lax.rem(iota + d, L) for d in range(L)]
    return iota, e


def _repack_tile(src, dst, iota, e, pe, je, rbl, cb):
    c_vec = cb * L + iota
    r_vecs = [rbl * L + e[d] for d in range(L)]
    vs = [plsc.load_gather(src, [c_vec, r_vecs[d]]) for d in range(L)]
    for d in range(L):
        plsc.store_scatter(
            dst, [rbl * (L // 2) + pe[d], c_vec + je[d]], vs[d]
        )


def _repack_kernel(V, D):
    RB = 2 * D
    n_blocks = V // RB
    steps = -(-n_blocks // NUM_WORKERS)
    steps += steps % 2
    mesh = plsc.VectorSubcoreMesh(core_axis_name="c", subcore_axis_name="s")

    @functools.partial(
        pl.kernel,
        mesh=mesh,
        out_type=jax.ShapeDtypeStruct((V // 2, 2 * D), jnp.float32),
        scratch_types=[
            pltpu.VMEM((2, D, RB), jnp.float32),
            pltpu.VMEM((2, RB // 2, 2 * D), jnp.float32),
        ]
        + [pltpu.SemaphoreType.DMA] * 4,
        compiler_params=pltpu.CompilerParams(needs_layout_passes=False),
    )
    def body(tt_hbm, tail_hbm, out_hbm, tin, tout, *sems):
        gsem = sems[:2]
        ssem = sems[2:]
        wid = lax.axis_index("s") * NUM_CORES + lax.axis_index("c")

        def block_id(k):
            return jnp.minimum(wid + k * NUM_WORKERS, n_blocks - 1)

        def load(k, b):
            r0 = pl.multiple_of(block_id(k) * RB, RB)
            return pltpu.make_async_copy(
                tt_hbm.at[:, pl.ds(r0, RB)], tin.at[b], gsem[b]
            )

        def store(k, b):
            p0 = pl.multiple_of(block_id(k) * (RB // 2), RB // 2)
            return pltpu.make_async_copy(
                tout.at[b], out_hbm.at[pl.ds(p0, RB // 2)], ssem[b]
            )

        load(0, 0).start()
        load(1, 1).start()

        iota, e = _diagonals()
        pe = [lax.shift_right_logical(e[d], 1) for d in range(L)]
        je = [lax.shift_left(jnp.bitwise_and(e[d], 1), 6) for d in range(L)]

        def step(i, _):
            for b in range(2):
                k = 2 * i + b
                load(k, b).wait()

                @pl.when(i > 0)
                def _():
                    store(k - 2, b).wait()

                def tile(t, _):
                    cb = t & (D // L - 1)
                    rbl = lax.shift_right_logical(t, 2)
                    _repack_tile(
                        tin.at[b], tout.at[b], iota, e, pe, je, rbl, cb
                    )
                    return 0

                lax.fori_loop(0, (D // L) * (RB // L), tile, 0)

                @pl.when(k + 2 < steps)
                def _():
                    load(k + 2, b).start()

                store(k, b).start()
            return 0

        lax.fori_loop(0, steps // 2, step, 0)
        store(steps - 2, 0).wait()
        store(steps - 1, 1).wait()

        @pl.when(wid == 0)
        def _():
            stage = tin.at[0, :, pl.ds(0, 2 * D)]
            pltpu.sync_copy(tail_hbm, stage)
            pltpu.sync_copy(stage, out_hbm.at[pl.ds(V // 2 - D, D)])

    return body


def _lookup_kernel(B0, B1, V, D):
    W = B0 // NUM_WORKERS
    NB = 4
    mesh = plsc.VectorSubcoreMesh(core_axis_name="c", subcore_axis_name="s")

    @functools.partial(
        pl.kernel,
        mesh=mesh,
        out_type=jax.ShapeDtypeStruct((B1, D, B0), jnp.float32),
        scratch_types=[
            pltpu.VMEM((B1, W), jnp.int32),
            pltpu.VMEM((NB, W), jnp.int32),
            pltpu.VMEM((NB, W), jnp.int32),
            pltpu.VMEM((NB, W, 2 * D), jnp.float32),
            pltpu.VMEM((2, D, W), jnp.float32),
        ]
        + [pltpu.SemaphoreType.DMA] * (NB + 2),
        compiler_params=pltpu.CompilerParams(needs_layout_passes=False),
    )
    def body(idx_hbm, tbl_hbm, out_hbm, idx_v, pidx, h64_v, grows, tout,
             *sems):
        gsem = sems[:NB]
        ssem = sems[NB:]
        wid = lax.axis_index("s") * NUM_CORES + lax.axis_index("c")
        c0 = pl.multiple_of(wid * W, W)
        pltpu.sync_copy(idx_hbm.at[:, pl.ds(c0, W)], idx_v)

        def prep(s, b):
            for g in range(W // L):
                r = idx_v[s, pl.ds(L * g, L)]
                pidx.at[b][pl.ds(L * g, L)] = lax.shift_right_logical(r, 1)
                h64_v.at[b][pl.ds(L * g, L)] = lax.shift_left(
                    jnp.bitwise_and(r, 1), 6
                )

        def gather(b):
            return pltpu.make_async_copy(
                tbl_hbm.at[pidx.at[b]], grows.at[b], gsem[b]
            )

        def start_gather(s, b):
            prep(s, b)
            gather(b).start()

        def store(s, b):
            return pltpu.make_async_copy(
                tout.at[b], out_hbm.at[s, :, pl.ds(c0, W)], ssem[b]
            )

        for b in range(NB):
            start_gather(b, b)

        iota, e = _diagonals()

        def step(i, _):
            for b in range(NB):
                s = NB * i + b
                tb = b % 2
                gather(b).wait()

                if b >= 2:
                    store(s - 2, tb).wait()
                else:

                    @pl.when(i > 0)
                    def _():
                        store(s - 2, tb).wait()

                def tile(t, _):
                    q = t & (D // L - 1)
                    g = lax.shift_right_logical(t, 2)
                    c_vec = q * L + iota
                    r_vecs = [g * L + e[d] for d in range(L)]
                    hs = [
                        plsc.load_gather(h64_v.at[b], [r_vecs[d]])
                        for d in range(L)
                    ]
                    vs = [
                        plsc.load_gather(
                            grows.at[b], [r_vecs[d], c_vec + hs[d]]
                        )
                        for d in range(L)
                    ]
                    for d in range(L):
                        plsc.store_scatter(
                            tout.at[tb], [c_vec, r_vecs[d]], vs[d]
                        )
                    return 0

                lax.fori_loop(0, (D // L) * (W // L), tile, 0)

                @pl.when(s + NB < B1)
                def _():
                    start_gather(s + NB, b)

                store(s, tb).start()
            return 0

        lax.fori_loop(0, B1 // NB, step, 0)
        store(B1 - 2, 0).wait()
        store(B1 - 1, 1).wait()

    return body


def kernel(indices, table):
    B0, B1 = indices.shape
    V, D = table.shape
    tt = table.T
    tail = table[V - 2 * D :].reshape(D, 2 * D)
    tbl = _repack_kernel(V, D)(tt, tail)
    idx2 = indices.T.astype(jnp.int32)
    out2 = _lookup_kernel(B0, B1, V, D)(idx2, tbl)
    return out2.transpose(2, 0, 1)


# speedup vs baseline: 1.7084x; 1.0800x over previous
import functools

import jax
import jax.numpy as jnp
from jax import lax
from jax.experimental import pallas as pl
from jax.experimental.pallas import tpu as pltpu
from jax.experimental.pallas import tpu_sc as plsc

NUM_CORES = 2
NUM_SUBCORES = 16
NUM_WORKERS = NUM_CORES * NUM_SUBCORES
L = 16


def _diagonals():
    iota = jnp.arange(L, dtype=jnp.int32)
    e = [lax.rem(iota + d, L) for d in range(L)]
    return iota, e


def _repack_tile(src, dst, iota, e, pe, je, rbl, cb):
    c_vec = cb * L + iota
    r_vecs = [rbl * L + e[d] for d in range(L)]
    vs = [plsc.load_gather(src, [c_vec, r_vecs[d]]) for d in range(L)]
    for d in range(L):
        plsc.store_scatter(
            dst, [rbl * (L // 2) + pe[d], c_vec + je[d]], vs[d]
        )


def _repack_kernel(V, D):
    RB = 2 * D
    n_blocks = V // RB
    NR = 4
    steps = -(-n_blocks // NUM_WORKERS)
    steps += (-steps) % NR
    mesh = plsc.VectorSubcoreMesh(core_axis_name="c", subcore_axis_name="s")

    @functools.partial(
        pl.kernel,
        mesh=mesh,
        out_type=jax.ShapeDtypeStruct((V // 2, 2 * D), jnp.float32),
        scratch_types=[
            pltpu.VMEM((NR, D, RB), jnp.float32),
            pltpu.VMEM((NR, RB // 2, 2 * D), jnp.float32),
        ]
        + [pltpu.SemaphoreType.DMA] * (2 * NR),
        compiler_params=pltpu.CompilerParams(needs_layout_passes=False),
    )
    def body(tt_hbm, tail_hbm, out_hbm, tin, tout, *sems):
        gsem = sems[:NR]
        ssem = sems[NR:]
        wid = lax.axis_index("s") * NUM_CORES + lax.axis_index("c")

        def block_id(k):
            return jnp.minimum(wid + k * NUM_WORKERS, n_blocks - 1)

        def load(k, b):
            r0 = pl.multiple_of(block_id(k) * RB, RB)
            return pltpu.make_async_copy(
                tt_hbm.at[:, pl.ds(r0, RB)], tin.at[b], gsem[b]
            )

        def store(k, b):
            p0 = pl.multiple_of(block_id(k) * (RB // 2), RB // 2)
            return pltpu.make_async_copy(
                tout.at[b], out_hbm.at[pl.ds(p0, RB // 2)], ssem[b]
            )

        for b in range(NR):
            load(b, b).start()

        iota, e = _diagonals()
        pe = [lax.shift_right_logical(e[d], 1) for d in range(L)]
        je = [lax.shift_left(jnp.bitwise_and(e[d], 1), 6) for d in range(L)]

        def step(i, _):
            for b in range(NR):
                k = NR * i + b
                load(k, b).wait()

                @pl.when(i > 0)
                def _():
                    store(k - NR, b).wait()

                def tile(t, _):
                    cb = t & (D // L - 1)
                    rbl = lax.shift_right_logical(t, 2)
                    _repack_tile(
                        tin.at[b], tout.at[b], iota, e, pe, je, rbl, cb
                    )
                    return 0

                lax.fori_loop(0, (D // L) * (RB // L), tile, 0)

                @pl.when(k + NR < steps)
                def _():
                    load(k + NR, b).start()

                store(k, b).start()
            return 0

        lax.fori_loop(0, steps // NR, step, 0)
        for b in range(NR):
            store(steps - NR + b, b).wait()

        @pl.when(wid == 0)
        def _():
            stage = tin.at[0, :, pl.ds(0, 2 * D)]
            pltpu.sync_copy(tail_hbm, stage)
            pltpu.sync_copy(stage, out_hbm.at[pl.ds(V // 2 - D, D)])

    return body


def _lookup_kernel(B0, B1, V, D):
    W = B0 // NUM_WORKERS
    NB = 4
    mesh = plsc.VectorSubcoreMesh(core_axis_name="c", subcore_axis_name="s")

    @functools.partial(
        pl.kernel,
        mesh=mesh,
        out_type=jax.ShapeDtypeStruct((B1, D, B0), jnp.float32),
        scratch_types=[
            pltpu.VMEM((B1, W), jnp.int32),
            pltpu.VMEM((NB, W), jnp.int32),
            pltpu.VMEM((NB, W), jnp.int32),
            pltpu.VMEM((NB, W, 2 * D), jnp.float32),
            pltpu.VMEM((NB, D, W), jnp.float32),
        ]
        + [pltpu.SemaphoreType.DMA] * (2 * NB),
        compiler_params=pltpu.CompilerParams(needs_layout_passes=False),
    )
    def body(idx_hbm, tbl_hbm, out_hbm, idx_v, pidx, h64_v, grows, tout,
             *sems):
        gsem = sems[:NB]
        ssem = sems[NB:]
        wid = lax.axis_index("s") * NUM_CORES + lax.axis_index("c")
        c0 = pl.multiple_of(wid * W, W)
        pltpu.sync_copy(idx_hbm.at[:, pl.ds(c0, W)], idx_v)

        def prep(s, b):
            for g in range(W // L):
                r = idx_v[s, pl.ds(L * g, L)]
                pidx.at[b][pl.ds(L * g, L)] = lax.shift_right_logical(r, 1)
                h64_v.at[b][pl.ds(L * g, L)] = lax.shift_left(
                    jnp.bitwise_and(r, 1), 6
                )

        def gather(b):
            return pltpu.make_async_copy(
                tbl_hbm.at[pidx.at[b]], grows.at[b], gsem[b]
            )

        def start_gather(s, b):
            prep(s, b)
            gather(b).start()

        def store(s, b):
            return pltpu.make_async_copy(
                tout.at[b], out_hbm.at[s, :, pl.ds(c0, W)], ssem[b]
            )

        for b in range(NB):
            start_gather(b, b)

        iota, e = _diagonals()

        def step(i, _):
            for b in range(NB):
                s = NB * i + b
                tb = b
                gather(b).wait()

                @pl.when(i > 0)
                def _():
                    store(s - NB, tb).wait()

                def tile(t, _):
                    q = t & (D // L - 1)
                    g = lax.shift_right_logical(t, 2)
                    c_vec = q * L + iota
                    r_vecs = [g * L + e[d] for d in range(L)]
                    hs = [
                        plsc.load_gather(h64_v.at[b], [r_vecs[d]])
                        for d in range(L)
                    ]
                    vs = [
                        plsc.load_gather(
                            grows.at[b], [r_vecs[d], c_vec + hs[d]]
                        )
                        for d in range(L)
                    ]
                    for d in range(L):
                        plsc.store_scatter(
                            tout.at[tb], [c_vec, r_vecs[d]], vs[d]
                        )
                    return 0

                lax.fori_loop(0, (D // L) * (W // L), tile, 0)

                @pl.when(s + NB < B1)
                def _():
                    start_gather(s + NB, b)

                store(s, tb).start()
            return 0

        lax.fori_loop(0, B1 // NB, step, 0)
        for b in range(NB):
            store(B1 - NB + b, b).wait()

    return body


def kernel(indices, table):
    B0, B1 = indices.shape
    V, D = table.shape
    tt = table.T
    tail = table[V - 2 * D :].reshape(D, 2 * D)
    tbl = _repack_kernel(V, D)(tt, tail)
    idx2 = indices.T.astype(jnp.int32)
    out2 = _lookup_kernel(B0, B1, V, D)(idx2, tbl)
    return out2.transpose(2, 0, 1)


# speedup vs baseline: 1.8145x; 1.0621x over previous
import functools

import jax
import jax.numpy as jnp
from jax import lax
from jax.experimental import pallas as pl
from jax.experimental.pallas import tpu as pltpu
from jax.experimental.pallas import tpu_sc as plsc

NUM_CORES = 2
NUM_SUBCORES = 16
NUM_WORKERS = NUM_CORES * NUM_SUBCORES
L = 16


def _diagonals():
    iota = jnp.arange(L, dtype=jnp.int32)
    e = [lax.rem(iota + d, L) for d in range(L)]
    return iota, e


def _repack_tile(src, dst, iota, e, pe, je, rbl, cb):
    c_vec = cb * L + iota
    r_vecs = [rbl * L + e[d] for d in range(L)]
    vs = [plsc.load_gather(src, [c_vec, r_vecs[d]]) for d in range(L)]
    for d in range(L):
        plsc.store_scatter(
            dst, [rbl * (L // 2) + pe[d], c_vec + je[d]], vs[d]
        )


def _repack_kernel(V, D):
    RB = 2 * D
    n_blocks = V // RB
    NR = 6
    steps = -(-n_blocks // NUM_WORKERS)
    steps += (-steps) % NR
    mesh = plsc.VectorSubcoreMesh(core_axis_name="c", subcore_axis_name="s")

    @functools.partial(
        pl.kernel,
        mesh=mesh,
        out_type=jax.ShapeDtypeStruct((V // 2, 2 * D), jnp.float32),
        scratch_types=[
            pltpu.VMEM((NR, D, RB), jnp.float32),
            pltpu.VMEM((NR, RB // 2, 2 * D), jnp.float32),
        ]
        + [pltpu.SemaphoreType.DMA] * (2 * NR),
        compiler_params=pltpu.CompilerParams(needs_layout_passes=False),
    )
    def body(tt_hbm, tail_hbm, out_hbm, tin, tout, *sems):
        gsem = sems[:NR]
        ssem = sems[NR:]
        wid = lax.axis_index("s") * NUM_CORES + lax.axis_index("c")

        def block_id(k):
            return jnp.minimum(wid + k * NUM_WORKERS, n_blocks - 1)

        def load(k, b):
            r0 = pl.multiple_of(block_id(k) * RB, RB)
            return pltpu.make_async_copy(
                tt_hbm.at[:, pl.ds(r0, RB)], tin.at[b], gsem[b]
            )

        def store(k, b):
            p0 = pl.multiple_of(block_id(k) * (RB // 2), RB // 2)
            return pltpu.make_async_copy(
                tout.at[b], out_hbm.at[pl.ds(p0, RB // 2)], ssem[b]
            )

        for b in range(NR):
            load(b, b).start()

        iota, e = _diagonals()
        pe = [lax.shift_right_logical(e[d], 1) for d in range(L)]
        je = [lax.shift_left(jnp.bitwise_and(e[d], 1), 6) for d in range(L)]

        def step(i, _):
            for b in range(NR):
                k = NR * i + b
                load(k, b).wait()

                @pl.when(i > 0)
                def _():
                    store(k - NR, b).wait()

                def tile(t, _):
                    cb = t & (D // L - 1)
                    rbl = lax.shift_right_logical(t, 2)
                    _repack_tile(
                        tin.at[b], tout.at[b], iota, e, pe, je, rbl, cb
                    )
                    return 0

                lax.fori_loop(0, (D // L) * (RB // L), tile, 0)

                @pl.when(k + NR < steps)
                def _():
                    load(k + NR, b).start()

                store(k, b).start()
            return 0

        lax.fori_loop(0, steps // NR, step, 0)
        for b in range(NR):
            store(steps - NR + b, b).wait()

        @pl.when(wid == 0)
        def _():
            stage = tin.at[0, :, pl.ds(0, 2 * D)]
            pltpu.sync_copy(tail_hbm, stage)
            pltpu.sync_copy(stage, out_hbm.at[pl.ds(V // 2 - D, D)])

    return body


def _lookup_kernel(B0, B1, V, D):
    W = B0 // NUM_WORKERS
    NB = 4
    mesh = plsc.VectorSubcoreMesh(core_axis_name="c", subcore_axis_name="s")

    @functools.partial(
        pl.kernel,
        mesh=mesh,
        out_type=jax.ShapeDtypeStruct((B1, D, B0), jnp.float32),
        scratch_types=[
            pltpu.VMEM((B1, W), jnp.int32),
            pltpu.VMEM((NB, W), jnp.int32),
            pltpu.VMEM((NB, W), jnp.int32),
            pltpu.VMEM((NB, W, 2 * D), jnp.float32),
            pltpu.VMEM((NB, D, W), jnp.float32),
        ]
        + [pltpu.SemaphoreType.DMA] * (2 * NB),
        compiler_params=pltpu.CompilerParams(needs_layout_passes=False),
    )
    def body(idx_hbm, tbl_hbm, out_hbm, idx_v, pidx, h64_v, grows, tout,
             *sems):
        gsem = sems[:NB]
        ssem = sems[NB:]
        wid = lax.axis_index("s") * NUM_CORES + lax.axis_index("c")
        c0 = pl.multiple_of(wid * W, W)
        pltpu.sync_copy(idx_hbm.at[:, pl.ds(c0, W)], idx_v)

        def prep(s, b):
            for g in range(W // L):
                r = idx_v[s, pl.ds(L * g, L)]
                pidx.at[b][pl.ds(L * g, L)] = lax.shift_right_logical(r, 1)
                h64_v.at[b][pl.ds(L * g, L)] = lax.shift_left(
                    jnp.bitwise_and(r, 1), 6
                )

        def gather(b):
            return pltpu.make_async_copy(
                tbl_hbm.at[pidx.at[b]], grows.at[b], gsem[b]
            )

        def start_gather(s, b):
            prep(s, b)
            gather(b).start()

        def store(s, b):
            return pltpu.make_async_copy(
                tout.at[b], out_hbm.at[s, :, pl.ds(c0, W)], ssem[b]
            )

        for b in range(NB):
            start_gather(b, b)

        iota, e = _diagonals()

        def step(i, _):
            for b in range(NB):
                s = NB * i + b
                tb = b
                gather(b).wait()

                @pl.when(i > 0)
                def _():
                    store(s - NB, tb).wait()

                def tile(t, _):
                    q = t & (D // L - 1)
                    g = lax.shift_right_logical(t, 2)
                    c_vec = q * L + iota
                    r_vecs = [g * L + e[d] for d in range(L)]
                    hs = [
                        plsc.load_gather(h64_v.at[b], [r_vecs[d]])
                        for d in range(L)
                    ]
                    vs = [
                        plsc.load_gather(
                            grows.at[b], [r_vecs[d], c_vec + hs[d]]
                        )
                        for d in range(L)
                    ]
                    for d in range(L):
                        plsc.store_scatter(
                            tout.at[tb], [c_vec, r_vecs[d]], vs[d]
                        )
                    return 0

                lax.fori_loop(0, (D // L) * (W // L), tile, 0)

                @pl.when(s + NB < B1)
                def _():
                    start_gather(s + NB, b)

                store(s, tb).start()
            return 0

        lax.fori_loop(0, B1 // NB, step, 0)
        for b in range(NB):
            store(B1 - NB + b, b).wait()

    return body


def kernel(indices, table):
    B0, B1 = indices.shape
    V, D = table.shape
    tt = table.T
    tail = table[V - 2 * D :].reshape(D, 2 * D)
    tbl = _repack_kernel(V, D)(tt, tail)
    idx2 = indices.T.astype(jnp.int32)
    out2 = _lookup_kernel(B0, B1, V, D)(idx2, tbl)
    return out2.transpose(2, 0, 1)


# speedup vs baseline: 1.8368x; 1.0123x over previous
import functools

import jax
import jax.numpy as jnp
from jax import lax
from jax.experimental import pallas as pl
from jax.experimental.pallas import tpu as pltpu
from jax.experimental.pallas import tpu_sc as plsc

NUM_CORES = 2
NUM_SUBCORES = 16
NUM_WORKERS = NUM_CORES * NUM_SUBCORES
L = 16


def _diagonals():
    iota = jnp.arange(L, dtype=jnp.int32)
    e = [lax.rem(iota + d, L) for d in range(L)]
    return iota, e


def _repack_tile(src, dst, iota, e, pe, je, rbl, cb):
    c_vec = cb * L + iota
    r_vecs = [rbl * L + e[d] for d in range(L)]
    vs = [plsc.load_gather(src, [c_vec, r_vecs[d]]) for d in range(L)]
    for d in range(L):
        plsc.store_scatter(
            dst, [rbl * (L // 2) + pe[d], c_vec + je[d]], vs[d]
        )


def _repack_kernel(V, D):
    RB = 2 * D
    n_blocks = V // RB
    NR = 7
    steps = -(-n_blocks // NUM_WORKERS)
    steps += (-steps) % NR
    mesh = plsc.VectorSubcoreMesh(core_axis_name="c", subcore_axis_name="s")

    @functools.partial(
        pl.kernel,
        mesh=mesh,
        out_type=jax.ShapeDtypeStruct((V // 2, 2 * D), jnp.float32),
        scratch_types=[
            pltpu.VMEM((NR, D, RB), jnp.float32),
            pltpu.VMEM((NR, RB // 2, 2 * D), jnp.float32),
        ]
        + [pltpu.SemaphoreType.DMA] * (2 * NR),
        compiler_params=pltpu.CompilerParams(needs_layout_passes=False),
    )
    def body(tt_hbm, tail_hbm, out_hbm, tin, tout, *sems):
        gsem = sems[:NR]
        ssem = sems[NR:]
        wid = lax.axis_index("s") * NUM_CORES + lax.axis_index("c")

        def block_id(k):
            return jnp.minimum(wid + k * NUM_WORKERS, n_blocks - 1)

        def load(k, b):
            r0 = pl.multiple_of(block_id(k) * RB, RB)
            return pltpu.make_async_copy(
                tt_hbm.at[:, pl.ds(r0, RB)], tin.at[b], gsem[b]
            )

        def store(k, b):
            p0 = pl.multiple_of(block_id(k) * (RB // 2), RB // 2)
            return pltpu.make_async_copy(
                tout.at[b], out_hbm.at[pl.ds(p0, RB // 2)], ssem[b]
            )

        for b in range(NR):
            load(b, b).start()

        iota, e = _diagonals()
        pe = [lax.shift_right_logical(e[d], 1) for d in range(L)]
        je = [lax.shift_left(jnp.bitwise_and(e[d], 1), 6) for d in range(L)]

        def step(i, _):
            for b in range(NR):
                k = NR * i + b
                load(k, b).wait()

                @pl.when(i > 0)
                def _():
                    store(k - NR, b).wait()

                def tile(t, _):
                    cb = t & (D // L - 1)
                    rbl = lax.shift_right_logical(t, 2)
                    _repack_tile(
                        tin.at[b], tout.at[b], iota, e, pe, je, rbl, cb
                    )
                    return 0

                lax.fori_loop(0, (D // L) * (RB // L), tile, 0)

                @pl.when(k + NR < steps)
                def _():
                    load(k + NR, b).start()

                store(k, b).start()
            return 0

        lax.fori_loop(0, steps // NR, step, 0)
        for b in range(NR):
            store(steps - NR + b, b).wait()

        @pl.when(wid == 0)
        def _():
            stage = tin.at[0, :, pl.ds(0, 2 * D)]
            pltpu.sync_copy(tail_hbm, stage)
            pltpu.sync_copy(stage, out_hbm.at[pl.ds(V // 2 - D, D)])

    return body


def _lookup_kernel(B0, B1, V, D):
    W = B0 // NUM_WORKERS
    NB = 4
    mesh = plsc.VectorSubcoreMesh(core_axis_name="c", subcore_axis_name="s")

    @functools.partial(
        pl.kernel,
        mesh=mesh,
        out_type=jax.ShapeDtypeStruct((B1, D, B0), jnp.float32),
        scratch_types=[
            pltpu.VMEM((B1, W), jnp.int32),
            pltpu.VMEM((NB, W), jnp.int32),
            pltpu.VMEM((NB, W), jnp.int32),
            pltpu.VMEM((NB, W, 2 * D), jnp.float32),
            pltpu.VMEM((NB, D, W), jnp.float32),
        ]
        + [pltpu.SemaphoreType.DMA] * (2 * NB),
        compiler_params=pltpu.CompilerParams(needs_layout_passes=False),
    )
    def body(idx_hbm, tbl_hbm, out_hbm, idx_v, pidx, h64_v, grows, tout,
             *sems):
        gsem = sems[:NB]
        ssem = sems[NB:]
        wid = lax.axis_index("s") * NUM_CORES + lax.axis_index("c")
        c0 = pl.multiple_of(wid * W, W)
        pltpu.sync_copy(idx_hbm.at[:, pl.ds(c0, W)], idx_v)

        def prep(s, b):
            for g in range(W // L):
                r = idx_v[s, pl.ds(L * g, L)]
                pidx.at[b][pl.ds(L * g, L)] = lax.shift_right_logical(r, 1)
                h64_v.at[b][pl.ds(L * g, L)] = lax.shift_left(
                    jnp.bitwise_and(r, 1), 6
                )

        def gather(b):
            return pltpu.make_async_copy(
                tbl_hbm.at[pidx.at[b]], grows.at[b], gsem[b]
            )

        def start_gather(s, b):
            prep(s, b)
            gather(b).start()

        def store(s, b):
            return pltpu.make_async_copy(
                tout.at[b], out_hbm.at[s, :, pl.ds(c0, W)], ssem[b]
            )

        for b in range(NB):
            start_gather(b, b)

        iota, e = _diagonals()

        def step(i, _):
            for b in range(NB):
                s = NB * i + b
                tb = b
                gather(b).wait()

                @pl.when(i > 0)
                def _():
                    store(s - NB, tb).wait()

                def tile(t, _):
                    q = t & (D // L - 1)
                    g = lax.shift_right_logical(t, 2)
                    c_vec = q * L + iota
                    r_vecs = [g * L + e[d] for d in range(L)]
                    hs = [
                        plsc.load_gather(h64_v.at[b], [r_vecs[d]])
                        for d in range(L)
                    ]
                    vs = [
                        plsc.load_gather(
                            grows.at[b], [r_vecs[d], c_vec + hs[d]]
                        )
                        for d in range(L)
                    ]
                    for d in range(L):
                        plsc.store_scatter(
                            tout.at[tb], [c_vec, r_vecs[d]], vs[d]
                        )
                    return 0

                lax.fori_loop(0, (D // L) * (W // L), tile, 0)

                @pl.when(s + NB < B1)
                def _():
                    start_gather(s + NB, b)

                store(s, tb).start()
            return 0

        lax.fori_loop(0, B1 // NB, step, 0)
        for b in range(NB):
            store(B1 - NB + b, b).wait()

    return body


def kernel(indices, table):
    B0, B1 = indices.shape
    V, D = table.shape
    tt = table.T
    tail = table[V - 2 * D :].reshape(D, 2 * D)
    tbl = _repack_kernel(V, D)(tt, tail)
    idx2 = indices.T.astype(jnp.int32)
    out2 = _lookup_kernel(B0, B1, V, D)(idx2, tbl)
    return out2.transpose(2, 0, 1)
